# baseline (speedup 1.0000x reference)
import functools

import jax
import jax.numpy as jnp
from jax import lax
from jax.experimental import pallas as pl
from jax.experimental.pallas import tpu as pltpu
from jax.experimental.pallas import tpu_sc as plsc

N_SRC = 100000
N_MID = 16000
N_DST = 4000
E0 = 256000
E1 = 64000
P = 4000
D = 256
H = 256

NC = 2
NS = 16
FH = 128
TR = 32
CHUNK = 125
R0 = E0 // CHUNK
R1 = E1 // CHUNK
C0 = R0 // NS
C1 = R1 // NS
PP = 4096

_f32 = jnp.float32


def _tc1_body(x_ref, wp_ref, bp_ref, q1_ref, bq1_ref, za_out, zb_out):
    h = jnp.dot(x_ref[...], wp_ref[...], preferred_element_type=_f32) + bp_ref[...]
    z = jnp.maximum(jnp.dot(h, q1_ref[...], preferred_element_type=_f32) + bq1_ref[...], 0.0)
    za_out[...] = z[:, :FH]
    zb_out[...] = z[:, FH:]


def _tc2_body(x_ref, aa_ref, ab_ref, deg_ref, wp_ref, bp_ref,
              w1_ref, bw1_ref, q2_ref, bq2_ref, h1_out, za_out, zb_out):
    h_dst = jnp.dot(x_ref[...], wp_ref[...], preferred_element_type=_f32) + bp_ref[...]
    deg = jnp.clip(deg_ref[...], 1.0, None)
    w1 = w1_ref[...]
    u = (jnp.dot(h_dst, w1[:256], preferred_element_type=_f32)
         + jnp.dot(aa_ref[...] / deg, w1[256:384], preferred_element_type=_f32)
         + jnp.dot(ab_ref[...] / deg, w1[384:512], preferred_element_type=_f32)
         + bw1_ref[...])
    h1 = jnp.maximum(u, 0.0)
    nrm = jnp.sqrt(jnp.sum(h1 * h1, axis=1, keepdims=True))
    h1 = h1 / jnp.clip(nrm, 1e-6, None)
    z2 = jnp.maximum(jnp.dot(h1, q2_ref[...], preferred_element_type=_f32) + bq2_ref[...], 0.0)
    h1_out[...] = h1
    za_out[...] = z2[:, :FH]
    zb_out[...] = z2[:, FH:]


def _tc3_body(x_ref, h1_ref, aa_ref, ab_ref, deg_ref, bias_ref, wp_ref, bp_ref,
              w2_ref, bw2_ref, gamma_ref, beta_ref, tu_out, tv_out):
    h_item_dst = jnp.dot(x_ref[...], wp_ref[...], preferred_element_type=_f32) + bp_ref[...]
    deg = jnp.clip(deg_ref[...], 1.0, None)
    w2 = w2_ref[...]
    u = (jnp.dot(h1_ref[...], w2[:256], preferred_element_type=_f32)
         + jnp.dot(aa_ref[...] / deg, w2[256:384], preferred_element_type=_f32)
         + jnp.dot(ab_ref[...] / deg, w2[384:512], preferred_element_type=_f32)
         + bw2_ref[...])
    h2 = jnp.maximum(u, 0.0)
    nrm = jnp.sqrt(jnp.sum(h2 * h2, axis=1, keepdims=True))
    h2 = h2 / jnp.clip(nrm, 1e-6, None)
    h = h_item_dst + h2
    mu = jnp.mean(h, axis=1, keepdims=True)
    var = jnp.mean((h - mu) * (h - mu), axis=1, keepdims=True)
    h = (h - mu) / jnp.sqrt(var + 1e-5) * gamma_ref[...] + beta_ref[...]
    col = lax.broadcasted_iota(jnp.int32, (N_DST, 128), 1)
    b = jnp.broadcast_to(bias_ref[...], (N_DST, 128))
    pad_u = jnp.where(col == 0, b, jnp.where(col == 1, 1.0, 0.0))
    pad_v = jnp.where(col == 0, 1.0, jnp.where(col == 1, b, 0.0))
    tu_out[...] = jnp.concatenate([h, pad_u], axis=1)
    tv_out[...] = jnp.concatenate([h, pad_v], axis=1)


def _tc4_body(pos_ref, neg_ref, loss_out, auc_out):
    p = pos_ref[...]
    n = neg_ref[...]
    loss_out[...] = (jnp.sum(jnp.maximum(n - p + 1.0, 0.0)) / P).reshape(1, 1)
    auc_out[...] = (jnp.sum((p > n).astype(_f32)) / P).reshape(1, 1)


@functools.lru_cache(maxsize=None)
def _sc_mesh():
    return plsc.VectorSubcoreMesh(core_axis_name="c", subcore_axis_name="s")


@functools.lru_cache(maxsize=None)
def _make_segsum(n_seg, n_chunk_rows):
    maxc = n_chunk_rows // NS
    half = n_seg // 2
    nw = half // 1000

    gsz = min(maxc, 64)

    @functools.partial(
        pl.kernel,
        out_type=(jax.ShapeDtypeStruct((n_seg, FH), _f32),
                  jax.ShapeDtypeStruct((n_seg, FH), _f32)),
        mesh=_sc_mesh(),
        scratch_types=[
            pltpu.VMEM_SHARED((half + TR, FH), _f32),
            pltpu.VMEM((gsz, CHUNK), jnp.int32),
            pltpu.VMEM((gsz, CHUNK), jnp.int32),
            pltpu.VMEM((16, 16), jnp.int32),
            pltpu.VMEM((CHUNK, FH), _f32),
            pltpu.VMEM((CHUNK, FH), _f32),
            pltpu.SemaphoreType.DMA,
            pltpu.SemaphoreType.DMA,
            pltpu.SemaphoreType.DMA,
            pltpu.SemaphoreType.DMA,
        ],
    )
    def segsum(ta, tb, slo, dlo, shi, dhi, cnts_hbm, zeros_hbm, oa, ob,
               acc, idx_s, idx_d, cb, rows_a, rows_b,
               sem_ga, sem_gb, sem_sa, sem_sb):
        c = lax.axis_index("c")
        s = lax.axis_index("s")
        pltpu.sync_copy(cnts_hbm, cb)

        def do_pass(table, es2, ed2, out, base, cnt_lane):
            nch = cb[s, pl.ds(0, 16)][cnt_lane]

            @pl.when(s < nw)
            def _():
                pltpu.sync_copy(zeros_hbm, acc.at[pl.ds(s * 1000, 1000)])

            plsc.subcore_barrier()

            for grp in range(maxc // gsz):
                g0 = grp * gsz
                ngc = jnp.clip(nch - g0, 0, gsz)

                @pl.when(ngc > 0)
                def _():
                    pltpu.sync_copy(es2.at[pl.ds(s * 128 + g0, gsz)], idx_s)
                    pltpu.sync_copy(ed2.at[pl.ds(s * 128 + g0, gsz)], idx_d)
                    pltpu.async_copy(table.at[idx_s.at[0]], rows_a, sem_ga)

                    @pl.when(ngc > 1)
                    def _():
                        pltpu.async_copy(table.at[idx_s.at[1]], rows_b, sem_gb)

                    def body(g, _):
                        j = 2 * g
                        pltpu.make_async_copy(table.at[idx_s.at[j]],
                                              rows_a, sem_ga).wait()
                        pltpu.async_copy(rows_a, acc.at[idx_d.at[j]],
                                         sem_sa, add=True)

                        @pl.when(j + 1 < ngc)
                        def _():
                            pltpu.make_async_copy(table.at[idx_s.at[j + 1]],
                                                  rows_b, sem_gb).wait()
                            pltpu.async_copy(rows_b, acc.at[idx_d.at[j + 1]],
                                             sem_sb, add=True)

                        @pl.when(j + 2 < ngc)
                        def _():
                            pltpu.make_async_copy(rows_a, acc.at[idx_d.at[0]],
                                                  sem_sa).wait()
                            pltpu.async_copy(table.at[idx_s.at[j + 2]],
                                             rows_a, sem_ga)

                        @pl.when(j + 3 < ngc)
                        def _():
                            pltpu.make_async_copy(rows_b, acc.at[idx_d.at[0]],
                                                  sem_sb).wait()
                            pltpu.async_copy(table.at[idx_s.at[j + 3]],
                                             rows_b, sem_gb)

                        return 0

                    lax.fori_loop(0, (ngc + 1) // 2, body, 0)
                    pltpu.make_async_copy(rows_a, acc.at[idx_d.at[0]],
                                          sem_sa).wait()

                    @pl.when(ngc > 1)
                    def _():
                        pltpu.make_async_copy(rows_b, acc.at[idx_d.at[0]],
                                              sem_sb).wait()

            plsc.subcore_barrier()

            @pl.when(s < nw)
            def _():
                pltpu.sync_copy(acc.at[pl.ds(s * 1000, 1000)],
                                out.at[pl.ds(base + s * 1000, 1000)])

            plsc.subcore_barrier()

        @pl.when(c == 0)
        def _():
            do_pass(ta, slo, dlo, oa, 0, 0)
            do_pass(ta, shi, dhi, oa, half, 1)

        @pl.when(c == 1)
        def _():
            do_pass(tb, slo, dlo, ob, 0, 0)
            do_pass(tb, shi, dhi, ob, half, 1)

    return segsum


N_MID_PAD = 16384
N_DST_PAD = 4096
CAP = 16000
CAPB = CAP + 128


def _prep_tile(es_flat, ed_flat, ed2, zeros_hbm, deg_out,
               oslo, odlo, oshi, odhi, cnt_out,
               dacc, idx2, ones, fs, fd, cs_lo, cd_lo, cs_hi, cd_hi,
               csh, cst, s, e_pt, halfn, n_pad, cpt):
    lane = lax.iota(jnp.int32, 16)
    pltpu.sync_copy(es_flat.at[pl.ds(s * e_pt, e_pt)], fs.at[pl.ds(0, e_pt)])
    pltpu.sync_copy(ed_flat.at[pl.ds(s * e_pt, e_pt)], fd.at[pl.ds(0, e_pt)])
    wb = n_pad // NS
    pltpu.sync_copy(ed2.at[pl.ds(s * cpt, cpt)], idx2.at[pl.ds(0, cpt)])
    pltpu.sync_copy(zeros_hbm.at[pl.ds(0, wb)], dacc.at[pl.ds(s * wb, wb)])
    plsc.subcore_barrier()

    def dbody(j, _):
        pltpu.sync_copy(ones.at[pl.ds(0, CHUNK)], dacc.at[idx2.at[j]], add=True)
        return 0

    lax.fori_loop(0, cpt, dbody, 0)
    plsc.subcore_barrier()
    pltpu.sync_copy(dacc.at[pl.ds(s * wb, wb)], deg_out.at[pl.ds(s * wb, wb)])

    def cbody(k, carry):
        cl, ch = carry
        vs = fs[pl.ds(k * 16, 16)]
        vd = fd[pl.ds(k * 16, 16)]
        m = vd < halfn
        nm = jnp.logical_not(m)
        plsc.store_compressed(cs_lo.at[pl.ds(cl, 16)], vs, mask=m)
        plsc.store_compressed(cd_lo.at[pl.ds(cl, 16)], vd, mask=m)
        plsc.store_compressed(cs_hi.at[pl.ds(ch, 16)], vs, mask=nm)
        plsc.store_compressed(cd_hi.at[pl.ds(ch, 16)], vd - halfn, mask=nm)
        nlo = plsc.all_reduce_population_count(m)[0]
        return (cl + nlo, ch + (16 - nlo))

    cl, ch = lax.fori_loop(0, e_pt // 16, cbody, (jnp.int32(0), jnp.int32(0)))
    zero16 = jnp.zeros((16,), jnp.int32)
    td = halfn + lane
    for t in range(8):
        cs_lo[pl.ds(cl + t * 16, 16)] = zero16
        cd_lo[pl.ds(cl + t * 16, 16)] = td
        cs_hi[pl.ds(ch + t * 16, 16)] = zero16
        cd_hi[pl.ds(ch + t * 16, 16)] = td
    nch_lo = (cl + (CHUNK - 1)) // CHUNK
    nch_hi = (ch + (CHUNK - 1)) // CHUNK
    cst[...] = jnp.where(lane == 0, nch_lo, jnp.where(lane == 1, nch_hi, 0))
    pltpu.sync_copy(cst, csh.at[s])
    plsc.subcore_barrier()

    @pl.when(s == 0)
    def _():
        pltpu.sync_copy(csh, cnt_out)

    wlen = ((e_pt + 127) // 128) * 128
    pltpu.sync_copy(cs_lo.at[pl.ds(0, wlen)], oslo.at[pl.ds(s * CAP, wlen)])
    pltpu.sync_copy(cd_lo.at[pl.ds(0, wlen)], odlo.at[pl.ds(s * CAP, wlen)])
    pltpu.sync_copy(cs_hi.at[pl.ds(0, wlen)], oshi.at[pl.ds(s * CAP, wlen)])
    pltpu.sync_copy(cd_hi.at[pl.ds(0, wlen)], odhi.at[pl.ds(s * CAP, wlen)])


@functools.lru_cache(maxsize=None)
def _make_sc_prep():
    ilist = lambda: jax.ShapeDtypeStruct((NS * CAP,), jnp.int32)

    @functools.partial(
        pl.kernel,
        out_type=(jax.ShapeDtypeStruct((N_MID_PAD,), _f32),
                  jax.ShapeDtypeStruct((N_DST_PAD,), _f32),
                  ilist(), ilist(), ilist(), ilist(),
                  ilist(), ilist(), ilist(), ilist(),
                  jax.ShapeDtypeStruct((16, 16), jnp.int32),
                  jax.ShapeDtypeStruct((16, 16), jnp.int32)),
        mesh=_sc_mesh(),
        compiler_params=pltpu.CompilerParams(needs_layout_passes=False),
        scratch_types=[
            pltpu.VMEM_SHARED((N_MID_PAD,), _f32),
            pltpu.VMEM_SHARED((16, 16), jnp.int32),
            pltpu.VMEM((C0, CHUNK), jnp.int32),
            pltpu.VMEM((128,), _f32),
            pltpu.VMEM((CAPB,), jnp.int32),
            pltpu.VMEM((CAPB,), jnp.int32),
            pltpu.VMEM((CAPB,), jnp.int32),
            pltpu.VMEM((CAPB,), jnp.int32),
            pltpu.VMEM((CAPB,), jnp.int32),
            pltpu.VMEM((CAPB,), jnp.int32),
            pltpu.VMEM((16,), jnp.int32),
        ],
    )
    def prep(e0s, e0d, e1s, e1d, e0d2, e1d2, zeros_hbm, ones_hbm,
             deg1, deg2, o0slo, o0dlo, o0shi, o0dhi,
             o1slo, o1dlo, o1shi, o1dhi, cnt0, cnt1,
             dacc, csh, idx2, ones, fs, fd, cs_lo, cd_lo, cs_hi, cd_hi, cst):
        c = lax.axis_index("c")
        s = lax.axis_index("s")
        pltpu.sync_copy(ones_hbm, ones)

        @pl.when(c == 0)
        def _():
            _prep_tile(e0s, e0d, e0d2, zeros_hbm, deg1,
                       o0slo, o0dlo, o0shi, o0dhi, cnt0,
                       dacc, idx2, ones, fs, fd, cs_lo, cd_lo, cs_hi, cd_hi,
                       csh, cst, s, E0 // NS, N_MID // 2, N_MID_PAD, C0)

        @pl.when(c == 1)
        def _():
            _prep_tile(e1s, e1d, e1d2, zeros_hbm, deg2,
                       o1slo, o1dlo, o1shi, o1dhi, cnt1,
                       dacc, idx2, ones, fs, fd, cs_lo, cd_lo, cs_hi, cd_hi,
                       csh, cst, s, E1 // NS, N_DST // 2, N_DST_PAD, C1)

    return prep


@functools.lru_cache(maxsize=None)
def _make_sc_score():
    @functools.partial(
        pl.kernel,
        out_type=(jax.ShapeDtypeStruct((PP,), _f32),
                  jax.ShapeDtypeStruct((PP,), _f32)),
        mesh=_sc_mesh(),
        compiler_params=pltpu.CompilerParams(needs_layout_passes=False),
        scratch_types=[
            pltpu.VMEM((128,), jnp.int32),
            pltpu.VMEM((128,), jnp.int32),
            pltpu.VMEM((128, 384), _f32),
            pltpu.VMEM((128, 384), _f32),
            pltpu.VMEM((128,), _f32),
            pltpu.SemaphoreType.DMA,
        ],
    )
    def _sc_score(tu_hbm, tv_hbm, pu_hbm, pv_hbm, nu_hbm, nv_hbm,
                  pos_out, neg_out, iu, iv, hu, hv, sc, sem):
        c = lax.axis_index("c")
        s = lax.axis_index("s")
        wid = s * NC + c
        base = wid * 128
        for u_hbm, v_hbm, out_hbm in ((pu_hbm, pv_hbm, pos_out),
                                      (nu_hbm, nv_hbm, neg_out)):
            pltpu.sync_copy(u_hbm.at[pl.ds(base, 128)], iu)
            pltpu.sync_copy(v_hbm.at[pl.ds(base, 128)], iv)
            pltpu.async_copy(tu_hbm.at[iu], hu, sem).wait()
            pltpu.async_copy(tv_hbm.at[iv], hv, sem).wait()
            lane = lax.iota(jnp.int32, 16)

            def body(q, _):
                vec = jnp.zeros((16,), _f32)
                for l in range(16):
                    p = q * 16 + l
                    acc = hu[p, pl.ds(0, 16)] * hv[p, pl.ds(0, 16)]
                    for f in range(1, 17):
                        acc = acc + hu[p, pl.ds(f * 16, 16)] * hv[p, pl.ds(f * 16, 16)]
                    vec = jnp.where(lane == l, jnp.sum(acc), vec)
                sc[pl.ds(q * 16, 16)] = vec
                return 0

            lax.fori_loop(0, 8, body, 0)
            pltpu.sync_copy(sc, out_hbm.at[pl.ds(base, 128)])

    return _sc_score


def kernel(x, e0_src, e0_dst, e1_src, e1_dst, pos_u, pos_v, neg_u, neg_v,
           W_proj, b_proj, Q1, bq1, W1, bw1, Q2, bq2, W2, bw2,
           item_bias, gamma, beta):
    bp = b_proj.reshape(1, H)
    bq1r = bq1.reshape(1, H)
    bw1r = bw1.reshape(1, H)
    bq2r = bq2.reshape(1, H)
    bw2r = bw2.reshape(1, H)
    gam = gamma.reshape(1, H)
    bet = beta.reshape(1, H)
    blk = 1000

    w_spec = [
        pl.BlockSpec((D, H), lambda i: (0, 0)),
        pl.BlockSpec((1, H), lambda i: (0, 0)),
    ]

    z1a, z1b = pl.pallas_call(
        _tc1_body,
        grid=(N_SRC // blk,),
        in_specs=[pl.BlockSpec((blk, D), lambda i: (i, 0))] + w_spec + w_spec,
        out_specs=[pl.BlockSpec((blk, FH), lambda i: (i, 0))] * 2,
        out_shape=[jax.ShapeDtypeStruct((N_SRC, FH), _f32)] * 2,
    )(x, W_proj, bp, Q1, bq1r)

    e0d2 = e0_dst.reshape(R0, CHUNK)
    e1d2 = e1_dst.reshape(R1, CHUNK)
    zeros1d = jnp.zeros((N_MID_PAD // NS,), _f32)
    ones1d = jnp.ones((128,), _f32)
    (deg1p, deg2p, o0slo, o0dlo, o0shi, o0dhi,
     o1slo, o1dlo, o1shi, o1dhi, cnt0, cnt1) = _make_sc_prep()(
        e0_src, e0_dst, e1_src, e1_dst, e0d2, e1d2, zeros1d, ones1d)
    deg1 = deg1p[:N_MID]
    deg2 = deg2p[:N_DST]
    rs = lambda a: a.reshape(NS * CAP // CHUNK, CHUNK)

    zeros_mid = jnp.zeros((1000, FH), _f32)
    acc1a, acc1b = _make_segsum(N_MID, R0)(
        z1a, z1b, rs(o0slo), rs(o0dlo), rs(o0shi), rs(o0dhi), cnt0, zeros_mid)

    h1, z2a, z2b = pl.pallas_call(
        _tc2_body,
        grid=(N_MID // blk,),
        in_specs=([pl.BlockSpec((blk, D), lambda i: (i, 0))]
                  + [pl.BlockSpec((blk, FH), lambda i: (i, 0))] * 2
                  + [pl.BlockSpec((blk, 1), lambda i: (i, 0))]
                  + w_spec
                  + [pl.BlockSpec((2 * H, H), lambda i: (0, 0)),
                     pl.BlockSpec((1, H), lambda i: (0, 0))]
                  + w_spec),
        out_specs=([pl.BlockSpec((blk, H), lambda i: (i, 0))]
                   + [pl.BlockSpec((blk, FH), lambda i: (i, 0))] * 2),
        out_shape=([jax.ShapeDtypeStruct((N_MID, H), _f32)]
                   + [jax.ShapeDtypeStruct((N_MID, FH), _f32)] * 2),
    )(x, acc1a, acc1b, deg1.reshape(N_MID, 1), W_proj, bp, W1, bw1r, Q2, bq2r)

    acc2a, acc2b = _make_segsum(N_DST, R1)(
        z2a, z2b, rs(o1slo), rs(o1dlo), rs(o1shi), rs(o1dhi), cnt1, zeros_mid)

    tu, tv = pl.pallas_call(
        _tc3_body,
        grid=(1,),
        in_specs=([pl.BlockSpec((N_DST, D), lambda i: (0, 0)),
                   pl.BlockSpec((N_DST, H), lambda i: (0, 0))]
                  + [pl.BlockSpec((N_DST, FH), lambda i: (0, 0))] * 2
                  + [pl.BlockSpec((N_DST, 1), lambda i: (0, 0)),
                     pl.BlockSpec((N_DST, 1), lambda i: (0, 0))]
                  + w_spec
                  + [pl.BlockSpec((2 * H, H), lambda i: (0, 0)),
                     pl.BlockSpec((1, H), lambda i: (0, 0)),
                     pl.BlockSpec((1, H), lambda i: (0, 0)),
                     pl.BlockSpec((1, H), lambda i: (0, 0))]),
        out_specs=[pl.BlockSpec((N_DST, 384), lambda i: (0, 0))] * 2,
        out_shape=[jax.ShapeDtypeStruct((N_DST, 384), _f32)] * 2,
    )(x, h1, acc2a, acc2b, deg2.reshape(N_DST, 1), item_bias.reshape(N_DST, 1),
      W_proj, bp, W2, bw2r, gam, bet)

    pad = jnp.zeros((PP - P,), jnp.int32)
    pu = jnp.concatenate([pos_u, pad])
    pv = jnp.concatenate([pos_v, pad])
    nu = jnp.concatenate([neg_u, pad])
    nv = jnp.concatenate([neg_v, pad])
    pos_raw, neg_raw = _make_sc_score()(tu, tv, pu, pv, nu, nv)
    pos_score = pos_raw[:P]
    neg_score = neg_raw[:P]

    loss2, auc2 = pl.pallas_call(
        _tc4_body,
        grid=(1,),
        in_specs=[pl.BlockSpec((8, 500), lambda i: (0, 0))] * 2,
        out_specs=[pl.BlockSpec((1, 1), lambda i: (0, 0))] * 2,
        out_shape=[jax.ShapeDtypeStruct((1, 1), _f32)] * 2,
    )(pos_score.reshape(8, 500), neg_score.reshape(8, 500))
    return (pos_score, neg_score, loss2.reshape(()), auc2.reshape(()))


# speedup vs baseline: 1.1000x; 1.1000x over previous
import functools

import jax
import jax.numpy as jnp
from jax import lax
from jax.experimental import pallas as pl
from jax.experimental.pallas import tpu as pltpu
from jax.experimental.pallas import tpu_sc as plsc

N_SRC = 100000
N_MID = 16000
N_DST = 4000
E0 = 256000
E1 = 64000
P = 4000
D = 256
H = 256

NC = 2
NS = 16
FH = 128
TR = 32
CHUNK = 125
R0 = E0 // CHUNK
R1 = E1 // CHUNK
C0 = R0 // NS
C1 = R1 // NS
PP = 4096

_f32 = jnp.float32


def _tc1_body(x_ref, wp_ref, bp_ref, q1_ref, bq1_ref, za_out, zb_out):
    h = jnp.dot(x_ref[...], wp_ref[...], preferred_element_type=_f32) + bp_ref[...]
    z = jnp.maximum(jnp.dot(h, q1_ref[...], preferred_element_type=_f32) + bq1_ref[...], 0.0)
    za_out[...] = z[:, :FH]
    zb_out[...] = z[:, FH:]


def _tc2_body(x_ref, aa_ref, ab_ref, deg_ref, wp_ref, bp_ref,
              w1_ref, bw1_ref, q2_ref, bq2_ref, h1_out, za_out, zb_out):
    h_dst = jnp.dot(x_ref[...], wp_ref[...], preferred_element_type=_f32) + bp_ref[...]
    deg = jnp.clip(deg_ref[...], 1.0, None)
    w1 = w1_ref[...]
    u = (jnp.dot(h_dst, w1[:256], preferred_element_type=_f32)
         + jnp.dot(aa_ref[...] / deg, w1[256:384], preferred_element_type=_f32)
         + jnp.dot(ab_ref[...] / deg, w1[384:512], preferred_element_type=_f32)
         + bw1_ref[...])
    h1 = jnp.maximum(u, 0.0)
    nrm = jnp.sqrt(jnp.sum(h1 * h1, axis=1, keepdims=True))
    h1 = h1 / jnp.clip(nrm, 1e-6, None)
    z2 = jnp.maximum(jnp.dot(h1, q2_ref[...], preferred_element_type=_f32) + bq2_ref[...], 0.0)
    h1_out[...] = h1
    za_out[...] = z2[:, :FH]
    zb_out[...] = z2[:, FH:]


def _tc3_body(x_ref, h1_ref, aa_ref, ab_ref, deg_ref, bias_ref, wp_ref, bp_ref,
              w2_ref, bw2_ref, gamma_ref, beta_ref, tu_out, tv_out):
    h_item_dst = jnp.dot(x_ref[...], wp_ref[...], preferred_element_type=_f32) + bp_ref[...]
    deg = jnp.clip(deg_ref[...], 1.0, None)
    w2 = w2_ref[...]
    u = (jnp.dot(h1_ref[...], w2[:256], preferred_element_type=_f32)
         + jnp.dot(aa_ref[...] / deg, w2[256:384], preferred_element_type=_f32)
         + jnp.dot(ab_ref[...] / deg, w2[384:512], preferred_element_type=_f32)
         + bw2_ref[...])
    h2 = jnp.maximum(u, 0.0)
    nrm = jnp.sqrt(jnp.sum(h2 * h2, axis=1, keepdims=True))
    h2 = h2 / jnp.clip(nrm, 1e-6, None)
    h = h_item_dst + h2
    mu = jnp.mean(h, axis=1, keepdims=True)
    var = jnp.mean((h - mu) * (h - mu), axis=1, keepdims=True)
    h = (h - mu) / jnp.sqrt(var + 1e-5) * gamma_ref[...] + beta_ref[...]
    col = lax.broadcasted_iota(jnp.int32, (N_DST, 128), 1)
    b = jnp.broadcast_to(bias_ref[...], (N_DST, 128))
    pad_u = jnp.where(col == 0, b, jnp.where(col == 1, 1.0, 0.0))
    pad_v = jnp.where(col == 0, 1.0, jnp.where(col == 1, b, 0.0))
    tu_out[...] = jnp.concatenate([h, pad_u], axis=1)
    tv_out[...] = jnp.concatenate([h, pad_v], axis=1)


def _tc4_body(pos_ref, neg_ref, loss_out, auc_out):
    p = pos_ref[...]
    n = neg_ref[...]
    loss_out[...] = (jnp.sum(jnp.maximum(n - p + 1.0, 0.0)) / P).reshape(1, 1)
    auc_out[...] = (jnp.sum((p > n).astype(_f32)) / P).reshape(1, 1)


@functools.lru_cache(maxsize=None)
def _sc_mesh():
    return plsc.VectorSubcoreMesh(core_axis_name="c", subcore_axis_name="s")


@functools.lru_cache(maxsize=None)
def _make_segsum(n_seg, n_chunk_rows):
    maxc = n_chunk_rows // NS
    half = n_seg // 2
    nw = half // 1000

    gsz = min(maxc, 64)

    @functools.partial(
        pl.kernel,
        out_type=(jax.ShapeDtypeStruct((n_seg, FH), _f32),
                  jax.ShapeDtypeStruct((n_seg, FH), _f32)),
        mesh=_sc_mesh(),
        scratch_types=[
            pltpu.VMEM_SHARED((half + TR, FH), _f32),
            pltpu.VMEM((gsz, CHUNK), jnp.int32),
            pltpu.VMEM((gsz, CHUNK), jnp.int32),
            pltpu.VMEM((16, 16), jnp.int32),
            pltpu.VMEM((CHUNK, FH), _f32),
            pltpu.VMEM((CHUNK, FH), _f32),
            pltpu.SemaphoreType.DMA,
            pltpu.SemaphoreType.DMA,
        ],
    )
    def segsum(ta, tb, slo, dlo, shi, dhi, cnts_hbm, zeros_hbm, oa, ob,
               acc, idx_s, idx_d, cb, rows_a, rows_b, sem_ga, sem_gb):
        c = lax.axis_index("c")
        s = lax.axis_index("s")
        pltpu.sync_copy(cnts_hbm, cb)

        def do_pass(table, es2, ed2, out, base, cnt_lane):
            nch = cb[s, pl.ds(0, 16)][cnt_lane]

            @pl.when(s < nw)
            def _():
                pltpu.sync_copy(zeros_hbm, acc.at[pl.ds(s * 1000, 1000)])

            plsc.subcore_barrier()

            for grp in range(maxc // gsz):
                g0 = grp * gsz
                ngc = jnp.clip(nch - g0, 0, gsz)

                @pl.when(ngc > 0)
                def _():
                    pltpu.sync_copy(es2.at[pl.ds(s * 128 + g0, gsz)], idx_s)
                    pltpu.sync_copy(ed2.at[pl.ds(s * 128 + g0, gsz)], idx_d)
                    pltpu.async_copy(table.at[idx_s.at[0]], rows_a, sem_ga)

                    def body(g, _):
                        j = 2 * g

                        @pl.when(j + 1 < ngc)
                        def _():
                            pltpu.async_copy(table.at[idx_s.at[j + 1]],
                                             rows_b, sem_gb)

                        pltpu.make_async_copy(table.at[idx_s.at[j]],
                                              rows_a, sem_ga).wait()
                        pltpu.sync_copy(rows_a, acc.at[idx_d.at[j]], add=True)

                        @pl.when(j + 2 < ngc)
                        def _():
                            pltpu.async_copy(table.at[idx_s.at[j + 2]],
                                             rows_a, sem_ga)

                        @pl.when(j + 1 < ngc)
                        def _():
                            pltpu.make_async_copy(table.at[idx_s.at[j + 1]],
                                                  rows_b, sem_gb).wait()
                            pltpu.sync_copy(rows_b, acc.at[idx_d.at[j + 1]],
                                            add=True)

                        return 0

                    lax.fori_loop(0, (ngc + 1) // 2, body, 0)

            plsc.subcore_barrier()

            @pl.when(s < nw)
            def _():
                pltpu.sync_copy(acc.at[pl.ds(s * 1000, 1000)],
                                out.at[pl.ds(base + s * 1000, 1000)])

            plsc.subcore_barrier()

        @pl.when(c == 0)
        def _():
            do_pass(ta, slo, dlo, oa, 0, 0)
            do_pass(ta, shi, dhi, oa, half, 1)

        @pl.when(c == 1)
        def _():
            do_pass(tb, slo, dlo, ob, 0, 0)
            do_pass(tb, shi, dhi, ob, half, 1)

    return segsum


N_MID_PAD = 16384
N_DST_PAD = 4096
CAP = 16000
CAPB = CAP + 128


def _prep_tile(es_flat, ed_flat, ed2, zeros_hbm, deg_out,
               oslo, odlo, oshi, odhi, cnt_out,
               dacc, idx2, ones, fs, fd, cs_lo, cd_lo, cs_hi, cd_hi,
               csh, cst, s, e_pt, halfn, n_pad, cpt):
    lane = lax.iota(jnp.int32, 16)
    pltpu.sync_copy(es_flat.at[pl.ds(s * e_pt, e_pt)], fs.at[pl.ds(0, e_pt)])
    pltpu.sync_copy(ed_flat.at[pl.ds(s * e_pt, e_pt)], fd.at[pl.ds(0, e_pt)])
    wb = n_pad // NS
    pltpu.sync_copy(ed2.at[pl.ds(s * cpt, cpt)], idx2.at[pl.ds(0, cpt)])
    pltpu.sync_copy(zeros_hbm.at[pl.ds(0, wb)], dacc.at[pl.ds(s * wb, wb)])
    plsc.subcore_barrier()

    def dbody(j, _):
        pltpu.sync_copy(ones.at[pl.ds(0, CHUNK)], dacc.at[idx2.at[j]], add=True)
        return 0

    lax.fori_loop(0, cpt, dbody, 0)
    plsc.subcore_barrier()
    pltpu.sync_copy(dacc.at[pl.ds(s * wb, wb)], deg_out.at[pl.ds(s * wb, wb)])

    def cbody(k, carry):
        cl, ch = carry
        vs = fs[pl.ds(k * 16, 16)]
        vd = fd[pl.ds(k * 16, 16)]
        m = vd < halfn
        nm = jnp.logical_not(m)
        plsc.store_compressed(cs_lo.at[pl.ds(cl, 16)], vs, mask=m)
        plsc.store_compressed(cd_lo.at[pl.ds(cl, 16)], vd, mask=m)
        plsc.store_compressed(cs_hi.at[pl.ds(ch, 16)], vs, mask=nm)
        plsc.store_compressed(cd_hi.at[pl.ds(ch, 16)], vd - halfn, mask=nm)
        nlo = plsc.all_reduce_population_count(m)[0]
        return (cl + nlo, ch + (16 - nlo))

    cl, ch = lax.fori_loop(0, e_pt // 16, cbody, (jnp.int32(0), jnp.int32(0)))
    zero16 = jnp.zeros((16,), jnp.int32)
    td = halfn + lane
    for t in range(8):
        cs_lo[pl.ds(cl + t * 16, 16)] = zero16
        cd_lo[pl.ds(cl + t * 16, 16)] = td
        cs_hi[pl.ds(ch + t * 16, 16)] = zero16
        cd_hi[pl.ds(ch + t * 16, 16)] = td
    nch_lo = (cl + (CHUNK - 1)) // CHUNK
    nch_hi = (ch + (CHUNK - 1)) // CHUNK
    cst[...] = jnp.where(lane == 0, nch_lo, jnp.where(lane == 1, nch_hi, 0))
    pltpu.sync_copy(cst, csh.at[s])
    plsc.subcore_barrier()

    @pl.when(s == 0)
    def _():
        pltpu.sync_copy(csh, cnt_out)

    wlen = ((e_pt + 127) // 128) * 128
    pltpu.sync_copy(cs_lo.at[pl.ds(0, wlen)], oslo.at[pl.ds(s * CAP, wlen)])
    pltpu.sync_copy(cd_lo.at[pl.ds(0, wlen)], odlo.at[pl.ds(s * CAP, wlen)])
    pltpu.sync_copy(cs_hi.at[pl.ds(0, wlen)], oshi.at[pl.ds(s * CAP, wlen)])
    pltpu.sync_copy(cd_hi.at[pl.ds(0, wlen)], odhi.at[pl.ds(s * CAP, wlen)])


@functools.lru_cache(maxsize=None)
def _make_sc_prep():
    ilist = lambda: jax.ShapeDtypeStruct((NS * CAP,), jnp.int32)

    @functools.partial(
        pl.kernel,
        out_type=(jax.ShapeDtypeStruct((N_MID_PAD,), _f32),
                  jax.ShapeDtypeStruct((N_DST_PAD,), _f32),
                  ilist(), ilist(), ilist(), ilist(),
                  ilist(), ilist(), ilist(), ilist(),
                  jax.ShapeDtypeStruct((16, 16), jnp.int32),
                  jax.ShapeDtypeStruct((16, 16), jnp.int32)),
        mesh=_sc_mesh(),
        compiler_params=pltpu.CompilerParams(needs_layout_passes=False),
        scratch_types=[
            pltpu.VMEM_SHARED((N_MID_PAD,), _f32),
            pltpu.VMEM_SHARED((16, 16), jnp.int32),
            pltpu.VMEM((C0, CHUNK), jnp.int32),
            pltpu.VMEM((128,), _f32),
            pltpu.VMEM((CAPB,), jnp.int32),
            pltpu.VMEM((CAPB,), jnp.int32),
            pltpu.VMEM((CAPB,), jnp.int32),
            pltpu.VMEM((CAPB,), jnp.int32),
            pltpu.VMEM((CAPB,), jnp.int32),
            pltpu.VMEM((CAPB,), jnp.int32),
            pltpu.VMEM((16,), jnp.int32),
        ],
    )
    def prep(e0s, e0d, e1s, e1d, e0d2, e1d2, zeros_hbm, ones_hbm,
             deg1, deg2, o0slo, o0dlo, o0shi, o0dhi,
             o1slo, o1dlo, o1shi, o1dhi, cnt0, cnt1,
             dacc, csh, idx2, ones, fs, fd, cs_lo, cd_lo, cs_hi, cd_hi, cst):
        c = lax.axis_index("c")
        s = lax.axis_index("s")
        pltpu.sync_copy(ones_hbm, ones)

        @pl.when(c == 0)
        def _():
            _prep_tile(e0s, e0d, e0d2, zeros_hbm, deg1,
                       o0slo, o0dlo, o0shi, o0dhi, cnt0,
                       dacc, idx2, ones, fs, fd, cs_lo, cd_lo, cs_hi, cd_hi,
                       csh, cst, s, E0 // NS, N_MID // 2, N_MID_PAD, C0)

        @pl.when(c == 1)
        def _():
            _prep_tile(e1s, e1d, e1d2, zeros_hbm, deg2,
                       o1slo, o1dlo, o1shi, o1dhi, cnt1,
                       dacc, idx2, ones, fs, fd, cs_lo, cd_lo, cs_hi, cd_hi,
                       csh, cst, s, E1 // NS, N_DST // 2, N_DST_PAD, C1)

    return prep


@functools.lru_cache(maxsize=None)
def _make_sc_score():
    @functools.partial(
        pl.kernel,
        out_type=(jax.ShapeDtypeStruct((PP,), _f32),
                  jax.ShapeDtypeStruct((PP,), _f32)),
        mesh=_sc_mesh(),
        compiler_params=pltpu.CompilerParams(needs_layout_passes=False),
        scratch_types=[
            pltpu.VMEM((128,), jnp.int32),
            pltpu.VMEM((128,), jnp.int32),
            pltpu.VMEM((128, 384), _f32),
            pltpu.VMEM((128, 384), _f32),
            pltpu.VMEM((128,), _f32),
            pltpu.SemaphoreType.DMA,
        ],
    )
    def _sc_score(tu_hbm, tv_hbm, pu_hbm, pv_hbm, nu_hbm, nv_hbm,
                  pos_out, neg_out, iu, iv, hu, hv, sc, sem):
        c = lax.axis_index("c")
        s = lax.axis_index("s")
        wid = s * NC + c
        base = wid * 128
        for u_hbm, v_hbm, out_hbm in ((pu_hbm, pv_hbm, pos_out),
                                      (nu_hbm, nv_hbm, neg_out)):
            pltpu.sync_copy(u_hbm.at[pl.ds(base, 128)], iu)
            pltpu.sync_copy(v_hbm.at[pl.ds(base, 128)], iv)
            pltpu.async_copy(tu_hbm.at[iu], hu, sem).wait()
            pltpu.async_copy(tv_hbm.at[iv], hv, sem).wait()
            lane = lax.iota(jnp.int32, 16)

            def body(q, _):
                vec = jnp.zeros((16,), _f32)
                for l in range(16):
                    p = q * 16 + l
                    acc = hu[p, pl.ds(0, 16)] * hv[p, pl.ds(0, 16)]
                    for f in range(1, 17):
                        acc = acc + hu[p, pl.ds(f * 16, 16)] * hv[p, pl.ds(f * 16, 16)]
                    vec = jnp.where(lane == l, jnp.sum(acc), vec)
                sc[pl.ds(q * 16, 16)] = vec
                return 0

            lax.fori_loop(0, 8, body, 0)
            pltpu.sync_copy(sc, out_hbm.at[pl.ds(base, 128)])

    return _sc_score


def kernel(x, e0_src, e0_dst, e1_src, e1_dst, pos_u, pos_v, neg_u, neg_v,
           W_proj, b_proj, Q1, bq1, W1, bw1, Q2, bq2, W2, bw2,
           item_bias, gamma, beta):
    bp = b_proj.reshape(1, H)
    bq1r = bq1.reshape(1, H)
    bw1r = bw1.reshape(1, H)
    bq2r = bq2.reshape(1, H)
    bw2r = bw2.reshape(1, H)
    gam = gamma.reshape(1, H)
    bet = beta.reshape(1, H)
    blk = 1000

    w_spec = [
        pl.BlockSpec((D, H), lambda i: (0, 0)),
        pl.BlockSpec((1, H), lambda i: (0, 0)),
    ]

    z1a, z1b = pl.pallas_call(
        _tc1_body,
        grid=(N_SRC // blk,),
        in_specs=[pl.BlockSpec((blk, D), lambda i: (i, 0))] + w_spec + w_spec,
        out_specs=[pl.BlockSpec((blk, FH), lambda i: (i, 0))] * 2,
        out_shape=[jax.ShapeDtypeStruct((N_SRC, FH), _f32)] * 2,
    )(x, W_proj, bp, Q1, bq1r)

    e0d2 = e0_dst.reshape(R0, CHUNK)
    e1d2 = e1_dst.reshape(R1, CHUNK)
    zeros1d = jnp.zeros((N_MID_PAD // NS,), _f32)
    ones1d = jnp.ones((128,), _f32)
    (deg1p, deg2p, o0slo, o0dlo, o0shi, o0dhi,
     o1slo, o1dlo, o1shi, o1dhi, cnt0, cnt1) = _make_sc_prep()(
        e0_src, e0_dst, e1_src, e1_dst, e0d2, e1d2, zeros1d, ones1d)
    deg1 = deg1p[:N_MID]
    deg2 = deg2p[:N_DST]
    rs = lambda a: a.reshape(NS * CAP // CHUNK, CHUNK)

    zeros_mid = jnp.zeros((1000, FH), _f32)
    acc1a, acc1b = _make_segsum(N_MID, R0)(
        z1a, z1b, rs(o0slo), rs(o0dlo), rs(o0shi), rs(o0dhi), cnt0, zeros_mid)

    h1, z2a, z2b = pl.pallas_call(
        _tc2_body,
        grid=(N_MID // blk,),
        in_specs=([pl.BlockSpec((blk, D), lambda i: (i, 0))]
                  + [pl.BlockSpec((blk, FH), lambda i: (i, 0))] * 2
                  + [pl.BlockSpec((blk, 1), lambda i: (i, 0))]
                  + w_spec
                  + [pl.BlockSpec((2 * H, H), lambda i: (0, 0)),
                     pl.BlockSpec((1, H), lambda i: (0, 0))]
                  + w_spec),
        out_specs=([pl.BlockSpec((blk, H), lambda i: (i, 0))]
                   + [pl.BlockSpec((blk, FH), lambda i: (i, 0))] * 2),
        out_shape=([jax.ShapeDtypeStruct((N_MID, H), _f32)]
                   + [jax.ShapeDtypeStruct((N_MID, FH), _f32)] * 2),
    )(x, acc1a, acc1b, deg1.reshape(N_MID, 1), W_proj, bp, W1, bw1r, Q2, bq2r)

    acc2a, acc2b = _make_segsum(N_DST, R1)(
        z2a, z2b, rs(o1slo), rs(o1dlo), rs(o1shi), rs(o1dhi), cnt1, zeros_mid)

    tu, tv = pl.pallas_call(
        _tc3_body,
        grid=(1,),
        in_specs=([pl.BlockSpec((N_DST, D), lambda i: (0, 0)),
                   pl.BlockSpec((N_DST, H), lambda i: (0, 0))]
                  + [pl.BlockSpec((N_DST, FH), lambda i: (0, 0))] * 2
                  + [pl.BlockSpec((N_DST, 1), lambda i: (0, 0)),
                     pl.BlockSpec((N_DST, 1), lambda i: (0, 0))]
                  + w_spec
                  + [pl.BlockSpec((2 * H, H), lambda i: (0, 0)),
                     pl.BlockSpec((1, H), lambda i: (0, 0)),
                     pl.BlockSpec((1, H), lambda i: (0, 0)),
                     pl.BlockSpec((1, H), lambda i: (0, 0))]),
        out_specs=[pl.BlockSpec((N_DST, 384), lambda i: (0, 0))] * 2,
        out_shape=[jax.ShapeDtypeStruct((N_DST, 384), _f32)] * 2,
    )(x, h1, acc2a, acc2b, deg2.reshape(N_DST, 1), item_bias.reshape(N_DST, 1),
      W_proj, bp, W2, bw2r, gam, bet)

    pad = jnp.zeros((PP - P,), jnp.int32)
    pu = jnp.concatenate([pos_u, pad])
    pv = jnp.concatenate([pos_v, pad])
    nu = jnp.concatenate([neg_u, pad])
    nv = jnp.concatenate([neg_v, pad])
    pos_raw, neg_raw = _make_sc_score()(tu, tv, pu, pv, nu, nv)
    pos_score = pos_raw[:P]
    neg_score = neg_raw[:P]

    loss2, auc2 = pl.pallas_call(
        _tc4_body,
        grid=(1,),
        in_specs=[pl.BlockSpec((8, 500), lambda i: (0, 0))] * 2,
        out_specs=[pl.BlockSpec((1, 1), lambda i: (0, 0))] * 2,
        out_shape=[jax.ShapeDtypeStruct((1, 1), _f32)] * 2,
    )(pos_score.reshape(8, 500), neg_score.reshape(8, 500))
    return (pos_score, neg_score, loss2.reshape(()), auc2.reshape(()))


# speedup vs baseline: 1.2490x; 1.1354x over previous
import functools

import jax
import jax.numpy as jnp
from jax import lax
from jax.experimental import pallas as pl
from jax.experimental.pallas import tpu as pltpu
from jax.experimental.pallas import tpu_sc as plsc

N_SRC = 100000
N_MID = 16000
N_DST = 4000
E0 = 256000
E1 = 64000
P = 4000
D = 256
H = 256

NC = 2
NS = 16
FH = 128
TR = 32
CHUNK = 125
R0 = E0 // CHUNK
R1 = E1 // CHUNK
C0 = R0 // NS
C1 = R1 // NS
PP = 4096

_f32 = jnp.float32


def _tc1_body(x_ref, wp_ref, bp_ref, q1_ref, bq1_ref, za_out, zb_out):
    h = jnp.dot(x_ref[...], wp_ref[...], preferred_element_type=_f32) + bp_ref[...]
    z = jnp.maximum(jnp.dot(h, q1_ref[...], preferred_element_type=_f32) + bq1_ref[...], 0.0)
    za_out[...] = z[:, :FH]
    zb_out[...] = z[:, FH:]


def _tc2_body(x_ref, aa_ref, ab_ref, deg_ref, wp_ref, bp_ref,
              w1_ref, bw1_ref, q2_ref, bq2_ref, h1_out, za_out, zb_out):
    h_dst = jnp.dot(x_ref[...], wp_ref[...], preferred_element_type=_f32) + bp_ref[...]
    deg = jnp.clip(deg_ref[...], 1.0, None)
    w1 = w1_ref[...]
    u = (jnp.dot(h_dst, w1[:256], preferred_element_type=_f32)
         + jnp.dot(aa_ref[...] / deg, w1[256:384], preferred_element_type=_f32)
         + jnp.dot(ab_ref[...] / deg, w1[384:512], preferred_element_type=_f32)
         + bw1_ref[...])
    h1 = jnp.maximum(u, 0.0)
    nrm = jnp.sqrt(jnp.sum(h1 * h1, axis=1, keepdims=True))
    h1 = h1 / jnp.clip(nrm, 1e-6, None)
    z2 = jnp.maximum(jnp.dot(h1, q2_ref[...], preferred_element_type=_f32) + bq2_ref[...], 0.0)
    h1_out[...] = h1
    za_out[...] = z2[:, :FH]
    zb_out[...] = z2[:, FH:]


def _tc3_body(x_ref, h1_ref, aa_ref, ab_ref, deg_ref, bias_ref, wp_ref, bp_ref,
              w2_ref, bw2_ref, gamma_ref, beta_ref, tu_out, tv_out):
    h_item_dst = jnp.dot(x_ref[...], wp_ref[...], preferred_element_type=_f32) + bp_ref[...]
    deg = jnp.clip(deg_ref[...], 1.0, None)
    w2 = w2_ref[...]
    u = (jnp.dot(h1_ref[...], w2[:256], preferred_element_type=_f32)
         + jnp.dot(aa_ref[...] / deg, w2[256:384], preferred_element_type=_f32)
         + jnp.dot(ab_ref[...] / deg, w2[384:512], preferred_element_type=_f32)
         + bw2_ref[...])
    h2 = jnp.maximum(u, 0.0)
    nrm = jnp.sqrt(jnp.sum(h2 * h2, axis=1, keepdims=True))
    h2 = h2 / jnp.clip(nrm, 1e-6, None)
    h = h_item_dst + h2
    mu = jnp.mean(h, axis=1, keepdims=True)
    var = jnp.mean((h - mu) * (h - mu), axis=1, keepdims=True)
    h = (h - mu) / jnp.sqrt(var + 1e-5) * gamma_ref[...] + beta_ref[...]
    col = lax.broadcasted_iota(jnp.int32, (N_DST, 128), 1)
    b = jnp.broadcast_to(bias_ref[...], (N_DST, 128))
    pad_u = jnp.where(col == 0, b, jnp.where(col == 1, 1.0, 0.0))
    pad_v = jnp.where(col == 0, 1.0, jnp.where(col == 1, b, 0.0))
    tu_out[...] = jnp.concatenate([h, pad_u], axis=1)
    tv_out[...] = jnp.concatenate([h, pad_v], axis=1)


def _tc4_body(pos_ref, neg_ref, loss_out, auc_out):
    p = pos_ref[...]
    n = neg_ref[...]
    loss_out[...] = (jnp.sum(jnp.maximum(n - p + 1.0, 0.0)) / P).reshape(1, 1)
    auc_out[...] = (jnp.sum((p > n).astype(_f32)) / P).reshape(1, 1)


@functools.lru_cache(maxsize=None)
def _sc_mesh():
    return plsc.VectorSubcoreMesh(core_axis_name="c", subcore_axis_name="s")


@functools.lru_cache(maxsize=None)
def _make_segsum(n_seg, n_chunk_rows):
    maxc = n_chunk_rows // NS
    half = n_seg // 2
    nw = half // 1000

    gsz = min(maxc, 64)

    @functools.partial(
        pl.kernel,
        out_type=(jax.ShapeDtypeStruct((n_seg, FH), _f32),
                  jax.ShapeDtypeStruct((n_seg, FH), _f32)),
        mesh=_sc_mesh(),
        scratch_types=[
            pltpu.VMEM_SHARED((half + TR, FH), _f32),
            pltpu.VMEM((gsz, CHUNK), jnp.int32),
            pltpu.VMEM((gsz, CHUNK), jnp.int32),
            pltpu.VMEM((16, 16), jnp.int32),
            pltpu.VMEM((CHUNK, FH), _f32),
            pltpu.VMEM((CHUNK, FH), _f32),
            pltpu.SemaphoreType.DMA,
            pltpu.SemaphoreType.DMA,
        ],
    )
    def segsum(ta, tb, slo, dlo, shi, dhi, cnts_hbm, zeros_hbm, oa, ob,
               acc, idx_s, idx_d, cb, rows_a, rows_b, sem_ga, sem_gb):
        c = lax.axis_index("c")
        s = lax.axis_index("s")
        pltpu.sync_copy(cnts_hbm, cb)

        def do_pass(table, es2, ed2, out, base, cnt_lane):
            nch = cb[s, pl.ds(0, 16)][cnt_lane]

            @pl.when(s < nw)
            def _():
                pltpu.sync_copy(zeros_hbm, acc.at[pl.ds(s * 1000, 1000)])

            plsc.subcore_barrier()

            for grp in range(maxc // gsz):
                g0 = grp * gsz
                ngc = jnp.clip(nch - g0, 0, gsz)

                @pl.when(ngc > 0)
                def _():
                    pltpu.sync_copy(es2.at[pl.ds(s * 128 + g0, gsz)], idx_s)
                    pltpu.sync_copy(ed2.at[pl.ds(s * 128 + g0, gsz)], idx_d)
                    pltpu.async_copy(table.at[idx_s.at[0]], rows_a, sem_ga)

                    def body(g, _):
                        j = 2 * g

                        @pl.when(j + 1 < ngc)
                        def _():
                            pltpu.async_copy(table.at[idx_s.at[j + 1]],
                                             rows_b, sem_gb)

                        pltpu.make_async_copy(table.at[idx_s.at[j]],
                                              rows_a, sem_ga).wait()
                        pltpu.sync_copy(rows_a, acc.at[idx_d.at[j]], add=True)

                        @pl.when(j + 2 < ngc)
                        def _():
                            pltpu.async_copy(table.at[idx_s.at[j + 2]],
                                             rows_a, sem_ga)

                        @pl.when(j + 1 < ngc)
                        def _():
                            pltpu.make_async_copy(table.at[idx_s.at[j + 1]],
                                                  rows_b, sem_gb).wait()
                            pltpu.sync_copy(rows_b, acc.at[idx_d.at[j + 1]],
                                            add=True)

                        return 0

                    lax.fori_loop(0, (ngc + 1) // 2, body, 0)

            plsc.subcore_barrier()

            @pl.when(s < nw)
            def _():
                pltpu.sync_copy(acc.at[pl.ds(s * 1000, 1000)],
                                out.at[pl.ds(base + s * 1000, 1000)])

            plsc.subcore_barrier()

        @pl.when(c == 0)
        def _():
            do_pass(ta, slo, dlo, oa, 0, 0)
            do_pass(ta, shi, dhi, oa, half, 1)

        @pl.when(c == 1)
        def _():
            do_pass(tb, slo, dlo, ob, 0, 0)
            do_pass(tb, shi, dhi, ob, half, 1)

    return segsum


@functools.lru_cache(maxsize=None)
def _make_segsum1(n_seg, n_chunk_rows):
    cpt = n_chunk_rows // NS
    nw = n_seg // 1000

    @functools.partial(
        pl.kernel,
        out_type=(jax.ShapeDtypeStruct((n_seg, FH), _f32),
                  jax.ShapeDtypeStruct((n_seg, FH), _f32)),
        mesh=_sc_mesh(),
        scratch_types=[
            pltpu.VMEM_SHARED((n_seg, FH), _f32),
            pltpu.VMEM((cpt, CHUNK), jnp.int32),
            pltpu.VMEM((cpt, CHUNK), jnp.int32),
            pltpu.VMEM((CHUNK, FH), _f32),
            pltpu.VMEM((CHUNK, FH), _f32),
            pltpu.SemaphoreType.DMA,
            pltpu.SemaphoreType.DMA,
        ],
    )
    def segsum1(ta, tb, es_hbm, ed_hbm, zeros_hbm, oa, ob,
                acc, idx_s, idx_d, rows_a, rows_b, sem_ga, sem_gb):
        c = lax.axis_index("c")
        s = lax.axis_index("s")
        pltpu.sync_copy(es_hbm.at[pl.ds(s * cpt, cpt)], idx_s)
        pltpu.sync_copy(ed_hbm.at[pl.ds(s * cpt, cpt)], idx_d)

        def run(table, out):
            @pl.when(s < nw)
            def _():
                pltpu.sync_copy(zeros_hbm, acc.at[pl.ds(s * 1000, 1000)])

            plsc.subcore_barrier()
            pltpu.async_copy(table.at[idx_s.at[0]], rows_a, sem_ga)

            def body(g, _):
                j = 2 * g

                @pl.when(j + 1 < cpt)
                def _():
                    pltpu.async_copy(table.at[idx_s.at[j + 1]], rows_b, sem_gb)

                pltpu.make_async_copy(table.at[idx_s.at[j]],
                                      rows_a, sem_ga).wait()
                pltpu.sync_copy(rows_a, acc.at[idx_d.at[j]], add=True)

                @pl.when(j + 2 < cpt)
                def _():
                    pltpu.async_copy(table.at[idx_s.at[j + 2]], rows_a, sem_ga)

                @pl.when(j + 1 < cpt)
                def _():
                    pltpu.make_async_copy(table.at[idx_s.at[j + 1]],
                                          rows_b, sem_gb).wait()
                    pltpu.sync_copy(rows_b, acc.at[idx_d.at[j + 1]], add=True)

                return 0

            lax.fori_loop(0, (cpt + 1) // 2, body, 0)
            plsc.subcore_barrier()

            @pl.when(s < nw)
            def _():
                pltpu.sync_copy(acc.at[pl.ds(s * 1000, 1000)],
                                out.at[pl.ds(s * 1000, 1000)])

        @pl.when(c == 0)
        def _():
            run(ta, oa)

        @pl.when(c == 1)
        def _():
            run(tb, ob)

    return segsum1


N_MID_PAD = 16384
N_DST_PAD = 4096
CAP = 16000
CAPB = CAP + 128


def _prep_tile(es_flat, ed_flat, ed2, zeros_hbm, deg_out,
               oslo, odlo, oshi, odhi, cnt_out,
               dacc, idx2, ones, fs, fd, cs_lo, cd_lo, cs_hi, cd_hi,
               csh, cst, s, e_pt, halfn, n_pad, cpt):
    lane = lax.iota(jnp.int32, 16)
    pltpu.sync_copy(es_flat.at[pl.ds(s * e_pt, e_pt)], fs.at[pl.ds(0, e_pt)])
    pltpu.sync_copy(ed_flat.at[pl.ds(s * e_pt, e_pt)], fd.at[pl.ds(0, e_pt)])
    wb = n_pad // NS
    pltpu.sync_copy(ed2.at[pl.ds(s * cpt, cpt)], idx2.at[pl.ds(0, cpt)])
    pltpu.sync_copy(zeros_hbm.at[pl.ds(0, wb)], dacc.at[pl.ds(s * wb, wb)])
    plsc.subcore_barrier()

    def dbody(j, _):
        pltpu.sync_copy(ones.at[pl.ds(0, CHUNK)], dacc.at[idx2.at[j]], add=True)
        return 0

    lax.fori_loop(0, cpt, dbody, 0)
    plsc.subcore_barrier()
    pltpu.sync_copy(dacc.at[pl.ds(s * wb, wb)], deg_out.at[pl.ds(s * wb, wb)])

    def cbody(k, carry):
        cl, ch = carry
        vs = fs[pl.ds(k * 16, 16)]
        vd = fd[pl.ds(k * 16, 16)]
        m = vd < halfn
        nm = jnp.logical_not(m)
        plsc.store_compressed(cs_lo.at[pl.ds(cl, 16)], vs, mask=m)
        plsc.store_compressed(cd_lo.at[pl.ds(cl, 16)], vd, mask=m)
        plsc.store_compressed(cs_hi.at[pl.ds(ch, 16)], vs, mask=nm)
        plsc.store_compressed(cd_hi.at[pl.ds(ch, 16)], vd - halfn, mask=nm)
        nlo = plsc.all_reduce_population_count(m)[0]
        return (cl + nlo, ch + (16 - nlo))

    cl, ch = lax.fori_loop(0, e_pt // 16, cbody, (jnp.int32(0), jnp.int32(0)))
    zero16 = jnp.zeros((16,), jnp.int32)
    td = halfn + lane
    for t in range(8):
        cs_lo[pl.ds(cl + t * 16, 16)] = zero16
        cd_lo[pl.ds(cl + t * 16, 16)] = td
        cs_hi[pl.ds(ch + t * 16, 16)] = zero16
        cd_hi[pl.ds(ch + t * 16, 16)] = td
    nch_lo = (cl + (CHUNK - 1)) // CHUNK
    nch_hi = (ch + (CHUNK - 1)) // CHUNK
    cst[...] = jnp.where(lane == 0, nch_lo, jnp.where(lane == 1, nch_hi, 0))
    pltpu.sync_copy(cst, csh.at[s])
    plsc.subcore_barrier()

    @pl.when(s == 0)
    def _():
        pltpu.sync_copy(csh, cnt_out)

    wlen = ((e_pt + 127) // 128) * 128
    pltpu.sync_copy(cs_lo.at[pl.ds(0, wlen)], oslo.at[pl.ds(s * CAP, wlen)])
    pltpu.sync_copy(cd_lo.at[pl.ds(0, wlen)], odlo.at[pl.ds(s * CAP, wlen)])
    pltpu.sync_copy(cs_hi.at[pl.ds(0, wlen)], oshi.at[pl.ds(s * CAP, wlen)])
    pltpu.sync_copy(cd_hi.at[pl.ds(0, wlen)], odhi.at[pl.ds(s * CAP, wlen)])


@functools.lru_cache(maxsize=None)
def _make_sc_prep():
    ilist = lambda: jax.ShapeDtypeStruct((NS * CAP,), jnp.int32)

    @functools.partial(
        pl.kernel,
        out_type=(jax.ShapeDtypeStruct((N_MID_PAD,), _f32),
                  jax.ShapeDtypeStruct((N_DST_PAD,), _f32),
                  ilist(), ilist(), ilist(), ilist(),
                  ilist(), ilist(), ilist(), ilist(),
                  jax.ShapeDtypeStruct((16, 16), jnp.int32),
                  jax.ShapeDtypeStruct((16, 16), jnp.int32)),
        mesh=_sc_mesh(),
        compiler_params=pltpu.CompilerParams(needs_layout_passes=False),
        scratch_types=[
            pltpu.VMEM_SHARED((N_MID_PAD,), _f32),
            pltpu.VMEM_SHARED((16, 16), jnp.int32),
            pltpu.VMEM((C0, CHUNK), jnp.int32),
            pltpu.VMEM((128,), _f32),
            pltpu.VMEM((CAPB,), jnp.int32),
            pltpu.VMEM((CAPB,), jnp.int32),
            pltpu.VMEM((CAPB,), jnp.int32),
            pltpu.VMEM((CAPB,), jnp.int32),
            pltpu.VMEM((CAPB,), jnp.int32),
            pltpu.VMEM((CAPB,), jnp.int32),
            pltpu.VMEM((16,), jnp.int32),
        ],
    )
    def prep(e0s, e0d, e1s, e1d, e0d2, e1d2, zeros_hbm, ones_hbm,
             deg1, deg2, o0slo, o0dlo, o0shi, o0dhi,
             o1slo, o1dlo, o1shi, o1dhi, cnt0, cnt1,
             dacc, csh, idx2, ones, fs, fd, cs_lo, cd_lo, cs_hi, cd_hi, cst):
        c = lax.axis_index("c")
        s = lax.axis_index("s")
        pltpu.sync_copy(ones_hbm, ones)

        @pl.when(c == 0)
        def _():
            _prep_tile(e0s, e0d, e0d2, zeros_hbm, deg1,
                       o0slo, o0dlo, o0shi, o0dhi, cnt0,
                       dacc, idx2, ones, fs, fd, cs_lo, cd_lo, cs_hi, cd_hi,
                       csh, cst, s, E0 // NS, N_MID // 2, N_MID_PAD, C0)

        @pl.when(c == 1)
        def _():
            _prep_tile(e1s, e1d, e1d2, zeros_hbm, deg2,
                       o1slo, o1dlo, o1shi, o1dhi, cnt1,
                       dacc, idx2, ones, fs, fd, cs_lo, cd_lo, cs_hi, cd_hi,
                       csh, cst, s, E1 // NS, N_DST // 2, N_DST_PAD, C1)

    return prep


@functools.lru_cache(maxsize=None)
def _make_sc_score():
    @functools.partial(
        pl.kernel,
        out_type=(jax.ShapeDtypeStruct((PP,), _f32),
                  jax.ShapeDtypeStruct((PP,), _f32)),
        mesh=_sc_mesh(),
        compiler_params=pltpu.CompilerParams(needs_layout_passes=False),
        scratch_types=[
            pltpu.VMEM((128,), jnp.int32),
            pltpu.VMEM((128,), jnp.int32),
            pltpu.VMEM((128, 384), _f32),
            pltpu.VMEM((128, 384), _f32),
            pltpu.VMEM((128,), _f32),
            pltpu.SemaphoreType.DMA,
        ],
    )
    def _sc_score(tu_hbm, tv_hbm, pu_hbm, pv_hbm, nu_hbm, nv_hbm,
                  pos_out, neg_out, iu, iv, hu, hv, sc, sem):
        c = lax.axis_index("c")
        s = lax.axis_index("s")
        wid = s * NC + c
        base = wid * 128
        for u_hbm, v_hbm, out_hbm in ((pu_hbm, pv_hbm, pos_out),
                                      (nu_hbm, nv_hbm, neg_out)):
            pltpu.sync_copy(u_hbm.at[pl.ds(base, 128)], iu)
            pltpu.sync_copy(v_hbm.at[pl.ds(base, 128)], iv)
            pltpu.async_copy(tu_hbm.at[iu], hu, sem).wait()
            pltpu.async_copy(tv_hbm.at[iv], hv, sem).wait()
            lane = lax.iota(jnp.int32, 16)

            def body(q, _):
                vec = jnp.zeros((16,), _f32)
                for l in range(16):
                    p = q * 16 + l
                    acc = hu[p, pl.ds(0, 16)] * hv[p, pl.ds(0, 16)]
                    for f in range(1, 17):
                        acc = acc + hu[p, pl.ds(f * 16, 16)] * hv[p, pl.ds(f * 16, 16)]
                    vec = jnp.where(lane == l, jnp.sum(acc), vec)
                sc[pl.ds(q * 16, 16)] = vec
                return 0

            lax.fori_loop(0, 8, body, 0)
            pltpu.sync_copy(sc, out_hbm.at[pl.ds(base, 128)])

    return _sc_score


def kernel(x, e0_src, e0_dst, e1_src, e1_dst, pos_u, pos_v, neg_u, neg_v,
           W_proj, b_proj, Q1, bq1, W1, bw1, Q2, bq2, W2, bw2,
           item_bias, gamma, beta):
    bp = b_proj.reshape(1, H)
    bq1r = bq1.reshape(1, H)
    bw1r = bw1.reshape(1, H)
    bq2r = bq2.reshape(1, H)
    bw2r = bw2.reshape(1, H)
    gam = gamma.reshape(1, H)
    bet = beta.reshape(1, H)
    blk = 1000

    w_spec = [
        pl.BlockSpec((D, H), lambda i: (0, 0)),
        pl.BlockSpec((1, H), lambda i: (0, 0)),
    ]

    z1a, z1b = pl.pallas_call(
        _tc1_body,
        grid=(N_SRC // blk,),
        in_specs=[pl.BlockSpec((blk, D), lambda i: (i, 0))] + w_spec + w_spec,
        out_specs=[pl.BlockSpec((blk, FH), lambda i: (i, 0))] * 2,
        out_shape=[jax.ShapeDtypeStruct((N_SRC, FH), _f32)] * 2,
    )(x, W_proj, bp, Q1, bq1r)

    e0d2 = e0_dst.reshape(R0, CHUNK)
    e1d2 = e1_dst.reshape(R1, CHUNK)
    zeros1d = jnp.zeros((N_MID_PAD // NS,), _f32)
    ones1d = jnp.ones((128,), _f32)
    (deg1p, deg2p, o0slo, o0dlo, o0shi, o0dhi,
     o1slo, o1dlo, o1shi, o1dhi, cnt0, cnt1) = _make_sc_prep()(
        e0_src, e0_dst, e1_src, e1_dst, e0d2, e1d2, zeros1d, ones1d)
    deg1 = deg1p[:N_MID]
    deg2 = deg2p[:N_DST]
    rs = lambda a: a.reshape(NS * CAP // CHUNK, CHUNK)

    zeros_mid = jnp.zeros((1000, FH), _f32)
    acc1a, acc1b = _make_segsum(N_MID, R0)(
        z1a, z1b, rs(o0slo), rs(o0dlo), rs(o0shi), rs(o0dhi), cnt0, zeros_mid)

    h1, z2a, z2b = pl.pallas_call(
        _tc2_body,
        grid=(N_MID // blk,),
        in_specs=([pl.BlockSpec((blk, D), lambda i: (i, 0))]
                  + [pl.BlockSpec((blk, FH), lambda i: (i, 0))] * 2
                  + [pl.BlockSpec((blk, 1), lambda i: (i, 0))]
                  + w_spec
                  + [pl.BlockSpec((2 * H, H), lambda i: (0, 0)),
                     pl.BlockSpec((1, H), lambda i: (0, 0))]
                  + w_spec),
        out_specs=([pl.BlockSpec((blk, H), lambda i: (i, 0))]
                   + [pl.BlockSpec((blk, FH), lambda i: (i, 0))] * 2),
        out_shape=([jax.ShapeDtypeStruct((N_MID, H), _f32)]
                   + [jax.ShapeDtypeStruct((N_MID, FH), _f32)] * 2),
    )(x, acc1a, acc1b, deg1.reshape(N_MID, 1), W_proj, bp, W1, bw1r, Q2, bq2r)

    e1s2 = e1_src.reshape(R1, CHUNK)
    acc2a, acc2b = _make_segsum1(N_DST, R1)(z2a, z2b, e1s2, e1d2, zeros_mid)

    tu, tv = pl.pallas_call(
        _tc3_body,
        grid=(1,),
        in_specs=([pl.BlockSpec((N_DST, D), lambda i: (0, 0)),
                   pl.BlockSpec((N_DST, H), lambda i: (0, 0))]
                  + [pl.BlockSpec((N_DST, FH), lambda i: (0, 0))] * 2
                  + [pl.BlockSpec((N_DST, 1), lambda i: (0, 0)),
                     pl.BlockSpec((N_DST, 1), lambda i: (0, 0))]
                  + w_spec
                  + [pl.BlockSpec((2 * H, H), lambda i: (0, 0)),
                     pl.BlockSpec((1, H), lambda i: (0, 0)),
                     pl.BlockSpec((1, H), lambda i: (0, 0)),
                     pl.BlockSpec((1, H), lambda i: (0, 0))]),
        out_specs=[pl.BlockSpec((N_DST, 384), lambda i: (0, 0))] * 2,
        out_shape=[jax.ShapeDtypeStruct((N_DST, 384), _f32)] * 2,
    )(x, h1, acc2a, acc2b, deg2.reshape(N_DST, 1), item_bias.reshape(N_DST, 1),
      W_proj, bp, W2, bw2r, gam, bet)

    pad = jnp.zeros((PP - P,), jnp.int32)
    pu = jnp.concatenate([pos_u, pad])
    pv = jnp.concatenate([pos_v, pad])
    nu = jnp.concatenate([neg_u, pad])
    nv = jnp.concatenate([neg_v, pad])
    pos_raw, neg_raw = _make_sc_score()(tu, tv, pu, pv, nu, nv)
    pos_score = pos_raw[:P]
    neg_score = neg_raw[:P]

    loss2, auc2 = pl.pallas_call(
        _tc4_body,
        grid=(1,),
        in_specs=[pl.BlockSpec((8, 500), lambda i: (0, 0))] * 2,
        out_specs=[pl.BlockSpec((1, 1), lambda i: (0, 0))] * 2,
        out_shape=[jax.ShapeDtypeStruct((1, 1), _f32)] * 2,
    )(pos_score.reshape(8, 500), neg_score.reshape(8, 500))
    return (pos_score, neg_score, loss2.reshape(()), auc2.reshape(()))
